# Initial kernel scaffold; baseline (speedup 1.0000x reference)
import jax
import jax.numpy as jnp
from jax.experimental import pallas as pl


def kernel(input_tensor_0, input_tensor_1, input_tensor_2, coordinate, W_r0, b_r0, W_r1, b_r1, W_r2, b_r2, idx_i, idx_j, atomic_number):
    raise NotImplementedError("write your pallas kernel here")


# speedup vs baseline: 37.9769x; 37.9769x over previous
import functools

import jax
import jax.numpy as jnp
from jax import lax
from jax.experimental import pallas as pl
from jax.experimental.pallas import tpu as pltpu
from jax.experimental.pallas import tpu_sc as plsc

N_ATOMS = 10000
N_EDGES = 160000
N_CHANNEL = 128
N_RBF = 16
R_CUT = 12.0
NORM_FACTOR = 16.0

D_TABLE = 14 * N_CHANNEL
D_COORD = 128
D_OUT = 13 * N_CHANNEL

SC_CORES = 2
SC_SUBCORES = 16
SC_WORKERS = SC_CORES * SC_SUBCORES

GW = 24
E_PAD = 161280
N_GBLK = E_PAD // GW

N_CHUNKS = 13
EDGES_PER_SUB = N_EDGES // SC_SUBCORES
SCAT_E = 80
SCAT_STEPS = EDGES_PER_SUB // SCAT_E
N_PAD = 10240
ROWS_PER_SUB = N_PAD // SC_SUBCORES

TC_B = 640


def _sc_mesh():
    return plsc.VectorSubcoreMesh(core_axis_name="c", subcore_axis_name="s")


def _gather_stage(table, coord16, idx_j, idx_i):
    pad = E_PAD - N_EDGES
    idx_j3 = jnp.pad(idx_j, (0, pad)).reshape(N_GBLK, 1, GW)
    idx_i3 = jnp.pad(idx_i, (0, pad)).reshape(N_GBLK, 1, GW)

    @functools.partial(
        pl.kernel,
        out_type=[
            jax.ShapeDtypeStruct((E_PAD, D_TABLE), jnp.float32),
            jax.ShapeDtypeStruct((E_PAD, D_COORD), jnp.float32),
        ],
        mesh=_sc_mesh(),
    )
    def gather_kernel(table_hbm, coord_hbm, ij_hbm, ii_hbm, g_hbm, ci_hbm):
        def body(ij_vmem, ii_vmem, g_vmem, ci_vmem):
            pltpu.sync_copy(table_hbm.at[ij_vmem.at[0, 0]], g_vmem)
            pltpu.sync_copy(coord_hbm.at[ii_vmem.at[0, 0]], ci_vmem)

        pltpu.emit_pipeline(
            body,
            grid=(N_GBLK,),
            in_specs=[
                pl.BlockSpec((1, 1, GW), lambda i: (i, 0, 0)),
                pl.BlockSpec((1, 1, GW), lambda i: (i, 0, 0)),
            ],
            out_specs=[
                pl.BlockSpec((GW, D_TABLE), lambda i: (i, 0)),
                pl.BlockSpec((GW, D_COORD), lambda i: (i, 0)),
            ],
            core_axis_name=("c", "s"),
            dimension_semantics=(pltpu.PARALLEL,),
        )(ij_hbm, ii_hbm, g_hbm, ci_hbm)

    return gather_kernel(table, coord16, idx_j3, idx_i3)


def _tc_body(g_ref, ci_ref, w0_ref, b0_ref, w1_ref, b1_ref, w2_ref, b2_ref,
             out_ref):
    g = g_ref[...]
    cj = g[:, 13 * N_CHANNEL:13 * N_CHANNEL + 3]
    ci = ci_ref[:, 0:3]
    r = cj - ci
    d2 = jnp.sum(r * r, axis=1, keepdims=True) + 1e-12
    d = jnp.sqrt(d2)
    u = r / d

    n = (lax.broadcasted_iota(jnp.int32, (1, N_RBF), 1) + 1).astype(jnp.float32)
    theta = n * (jnp.pi / R_CUT) * d
    rbf = jnp.sin(theta) / d
    dc = jnp.clip(d, 0.0, R_CUT)
    fc = 0.5 * (jnp.cos(dc * (jnp.pi / R_CUT)) + 1.0)
    fc = jnp.where(d < R_CUT, fc, 0.0)
    rbf = rbf * fc

    inv_norm = 1.0 / NORM_FACTOR
    f0 = (jnp.dot(rbf, w0_ref[...], preferred_element_type=jnp.float32)
          + b0_ref[...]) * inv_norm
    f1 = (jnp.dot(rbf, w1_ref[...], preferred_element_type=jnp.float32)
          + b1_ref[...]) * inv_norm
    f2 = (jnp.dot(rbf, w2_ref[...], preferred_element_type=jnp.float32)
          + b2_ref[...]) * inv_norm

    C = N_CHANNEL
    g0 = g[:, 0:C]
    g1 = [g[:, C + a * C: C + (a + 1) * C] for a in range(3)]
    g2 = [[g[:, 4 * C + (3 * a + b) * C: 4 * C + (3 * a + b + 1) * C]
           for b in range(3)] for a in range(3)]
    ub = [u[:, a:a + 1] for a in range(3)]

    d1 = g1[0] * ub[0] + g1[1] * ub[1] + g1[2] * ub[2]
    v2 = [g2[0][b] * ub[0] + g2[1][b] * ub[1] + g2[2][b] * ub[2]
          for b in range(3)]
    q2 = v2[0] * ub[0] + v2[1] * ub[1] + v2[2] * ub[2]

    out_ref[:, 0:C] = f0 * g0 + f1 * d1 + f2 * q2
    f1g0 = f1 * g0
    f2d1 = f2 * d1
    for b in range(3):
        out_ref[:, C + b * C: C + (b + 1) * C] = (
            f0 * g1[b] + f1g0 * ub[b] + f1 * v2[b] + f2d1 * ub[b])
    f2g0 = f2 * g0
    for a in range(3):
        ta = f2g0 * ub[a] + f1 * g1[a] + f2 * v2[a]
        for b in range(3):
            out_ref[:, 4 * C + (3 * a + b) * C: 4 * C + (3 * a + b + 1) * C] = (
                f0 * g2[a][b] + ta * ub[b])


def _tc_stage(gathered, ci, w0, b0, w1, b1, w2, b2):
    const_w = pl.BlockSpec((N_RBF, N_CHANNEL), lambda i: (0, 0))
    const_b = pl.BlockSpec((1, N_CHANNEL), lambda i: (0, 0))
    return pl.pallas_call(
        _tc_body,
        grid=(E_PAD // TC_B,),
        in_specs=[
            pl.BlockSpec((TC_B, D_TABLE), lambda i: (i, 0)),
            pl.BlockSpec((TC_B, D_COORD), lambda i: (i, 0)),
            const_w, const_b, const_w, const_b, const_w, const_b,
        ],
        out_specs=pl.BlockSpec((TC_B, D_OUT), lambda i: (i, 0)),
        out_shape=jax.ShapeDtypeStruct((E_PAD, D_OUT), jnp.float32),
    )(gathered, ci, w0, b0.reshape(1, N_CHANNEL), w1,
      b1.reshape(1, N_CHANNEL), w2, b2.reshape(1, N_CHANNEL))


def _scatter_stage(edge_out, idx_i, zeros_rows):
    idx_i3 = idx_i.reshape(N_EDGES // SCAT_E, 1, SCAT_E)

    @functools.partial(
        pl.kernel,
        out_type=jax.ShapeDtypeStruct((N_PAD, D_OUT), jnp.float32),
        mesh=_sc_mesh(),
        scratch_types=[
            pltpu.VMEM_SHARED((N_PAD, N_CHANNEL), jnp.float32),
            pltpu.VMEM((SCAT_E, N_CHANNEL), jnp.float32),
            pltpu.VMEM((1, SCAT_E), jnp.int32),
        ],
    )
    def scatter_kernel(eo_hbm, ii_hbm, z_hbm, out_hbm, acc, buf, idxb):
        core = lax.axis_index("c")
        sid = lax.axis_index("s")
        ebase = sid * EDGES_PER_SUB
        ibase = sid * SCAT_STEPS
        rbase = sid * ROWS_PER_SUB

        for k in range(N_CHUNKS):
            @pl.when(core == (k % SC_CORES))
            def _(k=k):
                pltpu.sync_copy(z_hbm, acc.at[pl.ds(rbase, ROWS_PER_SUB)])
                plsc.subcore_barrier()

                @pl.loop(0, SCAT_STEPS)
                def _(step):
                    e0 = ebase + step * SCAT_E
                    pltpu.sync_copy(
                        eo_hbm.at[pl.ds(e0, SCAT_E),
                                  pl.ds(k * N_CHANNEL, N_CHANNEL)],
                        buf)
                    pltpu.sync_copy(ii_hbm.at[ibase + step], idxb)
                    pltpu.sync_copy(buf, acc.at[idxb.at[0]], add=True)

                plsc.subcore_barrier()
                pltpu.sync_copy(
                    acc.at[pl.ds(rbase, ROWS_PER_SUB)],
                    out_hbm.at[pl.ds(rbase, ROWS_PER_SUB),
                               pl.ds(k * N_CHANNEL, N_CHANNEL)])

    return scatter_kernel(edge_out, idx_i3, zeros_rows)


def kernel(input_tensor_0, input_tensor_1, input_tensor_2, coordinate,
           W_r0, b_r0, W_r1, b_r1, W_r2, b_r2, idx_i, idx_j, atomic_number):
    n = N_ATOMS
    t1t = input_tensor_1.transpose(0, 2, 1).reshape(n, 3 * N_CHANNEL)
    t2t = input_tensor_2.transpose(0, 2, 3, 1).reshape(n, 9 * N_CHANNEL)
    coordp = jnp.pad(coordinate, ((0, 0), (0, D_COORD - 3)))
    table = jnp.concatenate([input_tensor_0, t1t, t2t, coordp], axis=1)

    gathered, ci = _gather_stage(table, coordp, idx_j, idx_i)
    edge_out = _tc_stage(gathered, ci, W_r0, b_r0, W_r1, b_r1, W_r2, b_r2)
    zeros_rows = jnp.zeros((ROWS_PER_SUB, N_CHANNEL), jnp.float32)
    out_flat = _scatter_stage(edge_out, idx_i, zeros_rows)

    out_flat = out_flat[:n]
    out0 = out_flat[:, 0:N_CHANNEL]
    out1 = out_flat[:, N_CHANNEL:4 * N_CHANNEL].reshape(
        n, 3, N_CHANNEL).transpose(0, 2, 1)
    out2 = out_flat[:, 4 * N_CHANNEL:].reshape(
        n, 3, 3, N_CHANNEL).transpose(0, 3, 1, 2)
    return (out0, out1, out2)


# speedup vs baseline: 56.8143x; 1.4960x over previous
import functools

import jax
import jax.numpy as jnp
from jax import lax
from jax.experimental import pallas as pl
from jax.experimental.pallas import tpu as pltpu
from jax.experimental.pallas import tpu_sc as plsc

N_ATOMS = 10000
N_EDGES = 160000
N_CHANNEL = 128
N_RBF = 16
R_CUT = 12.0
NORM_FACTOR = 16.0

D_TABLE = 14 * N_CHANNEL
D_COORD = 128
D_OUT = 13 * N_CHANNEL

SC_CORES = 2
SC_SUBCORES = 16
SC_WORKERS = SC_CORES * SC_SUBCORES

GW = 24
E_PAD = 161280
N_GBLK = E_PAD // GW

N_CHUNKS = 13
EDGES_PER_SUB = N_EDGES // SC_SUBCORES
SCAT_E = 128
SCAT_TOTAL_STEPS = N_EDGES // SCAT_E
SCAT_OUTER = 40
N_PAD = 10240
ROWS_PER_SUB = N_PAD // SC_SUBCORES

TC_B = 640


def _sc_mesh():
    return plsc.VectorSubcoreMesh(core_axis_name="c", subcore_axis_name="s")


def _gather_stage(table, coord16, idx_j, idx_i):
    pad = E_PAD - N_EDGES
    idx_j3 = jnp.pad(idx_j, (0, pad)).reshape(N_GBLK, 1, GW)
    idx_i3 = jnp.pad(idx_i, (0, pad)).reshape(N_GBLK, 1, GW)

    @functools.partial(
        pl.kernel,
        out_type=[
            jax.ShapeDtypeStruct((E_PAD, D_TABLE), jnp.float32),
            jax.ShapeDtypeStruct((E_PAD, D_COORD), jnp.float32),
        ],
        mesh=_sc_mesh(),
    )
    def gather_kernel(table_hbm, coord_hbm, ij_hbm, ii_hbm, g_hbm, ci_hbm):
        def body(ij_vmem, ii_vmem, g_vmem, ci_vmem):
            pltpu.sync_copy(table_hbm.at[ij_vmem.at[0, 0]], g_vmem)
            pltpu.sync_copy(coord_hbm.at[ii_vmem.at[0, 0]], ci_vmem)

        pltpu.emit_pipeline(
            body,
            grid=(N_GBLK,),
            in_specs=[
                pl.BlockSpec((1, 1, GW), lambda i: (i, 0, 0)),
                pl.BlockSpec((1, 1, GW), lambda i: (i, 0, 0)),
            ],
            out_specs=[
                pl.BlockSpec((GW, D_TABLE), lambda i: (i, 0)),
                pl.BlockSpec((GW, D_COORD), lambda i: (i, 0)),
            ],
            core_axis_name=("c", "s"),
            dimension_semantics=(pltpu.PARALLEL,),
        )(ij_hbm, ii_hbm, g_hbm, ci_hbm)

    return gather_kernel(table, coord16, idx_j3, idx_i3)


def _sin_turns(x):
    r = jnp.round(x)
    u = 2.0 * (x - r)
    w = u * u
    p = jnp.float32(-0.00614081537025124)
    p = p * w + jnp.float32(0.08086610273610417)
    p = p * w + jnp.float32(-0.5986449462113246)
    p = p * w + jnp.float32(2.5500285525212325)
    p = p * w + jnp.float32(-5.167702003410101)
    p = p * w + jnp.float32(3.1415925159778033)
    return p * u


def _tc_body(g_ref, ci_ref, w0_ref, b0_ref, w1_ref, b1_ref, w2_ref, b2_ref,
             out_ref):
    g = g_ref[...]
    cj = g[:, 13 * N_CHANNEL:13 * N_CHANNEL + 3]
    ci = ci_ref[:, 0:3]
    r = cj - ci
    d2 = jnp.sum(r * r, axis=1, keepdims=True) + 1e-12
    d = jnp.sqrt(d2)
    u = r / d

    n = (lax.broadcasted_iota(jnp.int32, (1, N_RBF), 1) + 1).astype(jnp.float32)
    xs = d * (n * (0.5 / R_CUT))
    xc = d * (0.5 / R_CUT) + 0.25
    sn = _sin_turns(xs)
    c1 = _sin_turns(xc)
    fc = jnp.where(d < R_CUT, 0.5 * (c1 + 1.0), 0.0)
    rbf = sn * (fc / d)

    inv_norm = 1.0 / NORM_FACTOR
    f0 = (jnp.dot(rbf, w0_ref[...], preferred_element_type=jnp.float32)
          + b0_ref[...]) * inv_norm
    f1 = (jnp.dot(rbf, w1_ref[...], preferred_element_type=jnp.float32)
          + b1_ref[...]) * inv_norm
    f2 = (jnp.dot(rbf, w2_ref[...], preferred_element_type=jnp.float32)
          + b2_ref[...]) * inv_norm

    C = N_CHANNEL
    g0 = g[:, 0:C]
    g1 = [g[:, C + a * C: C + (a + 1) * C] for a in range(3)]
    g2 = [[g[:, 4 * C + (3 * a + b) * C: 4 * C + (3 * a + b + 1) * C]
           for b in range(3)] for a in range(3)]
    ub = [u[:, a:a + 1] for a in range(3)]

    d1 = g1[0] * ub[0] + g1[1] * ub[1] + g1[2] * ub[2]
    v2 = [g2[0][b] * ub[0] + g2[1][b] * ub[1] + g2[2][b] * ub[2]
          for b in range(3)]
    q2 = v2[0] * ub[0] + v2[1] * ub[1] + v2[2] * ub[2]

    out_ref[0, :, :] = f0 * g0 + f1 * d1 + f2 * q2
    f1g0 = f1 * g0
    f2d1 = f2 * d1
    for b in range(3):
        out_ref[1 + b, :, :] = (
            f0 * g1[b] + f1g0 * ub[b] + f1 * v2[b] + f2d1 * ub[b])
    f2g0 = f2 * g0
    for a in range(3):
        ta = f2g0 * ub[a] + f1 * g1[a] + f2 * v2[a]
        for b in range(3):
            out_ref[4 + 3 * a + b, :, :] = f0 * g2[a][b] + ta * ub[b]


def _tc_stage(gathered, ci, w0, b0, w1, b1, w2, b2):
    const_w = pl.BlockSpec((N_RBF, N_CHANNEL), lambda i: (0, 0))
    const_b = pl.BlockSpec((1, N_CHANNEL), lambda i: (0, 0))
    return pl.pallas_call(
        _tc_body,
        grid=(E_PAD // TC_B,),
        in_specs=[
            pl.BlockSpec((TC_B, D_TABLE), lambda i: (i, 0)),
            pl.BlockSpec((TC_B, D_COORD), lambda i: (i, 0)),
            const_w, const_b, const_w, const_b, const_w, const_b,
        ],
        out_specs=pl.BlockSpec((N_CHUNKS, TC_B, N_CHANNEL), lambda i: (0, i, 0)),
        out_shape=jax.ShapeDtypeStruct((N_CHUNKS, E_PAD, N_CHANNEL), jnp.float32),
    )(gathered, ci, w0, b0.reshape(1, N_CHANNEL), w1,
      b1.reshape(1, N_CHANNEL), w2, b2.reshape(1, N_CHANNEL))


def _scatter_stage(edge_out, idx_i, zeros_rows):
    idx_i3 = idx_i.reshape(N_EDGES // SCAT_E, 1, SCAT_E)

    @functools.partial(
        pl.kernel,
        out_type=jax.ShapeDtypeStruct((N_CHUNKS, N_PAD, N_CHANNEL), jnp.float32),
        mesh=_sc_mesh(),
        scratch_types=[
            pltpu.VMEM_SHARED((N_PAD, N_CHANNEL), jnp.float32),
            pltpu.VMEM((SCAT_E, N_CHANNEL), jnp.float32),
            pltpu.VMEM((SCAT_E, N_CHANNEL), jnp.float32),
            pltpu.VMEM((1, SCAT_E), jnp.int32),
            pltpu.VMEM((1, SCAT_E), jnp.int32),
            pltpu.SemaphoreType.DMA,
            pltpu.SemaphoreType.DMA,
        ],
    )
    def scatter_kernel(eo_hbm, ii_hbm, z_hbm, out_hbm, acc,
                       buf0, buf1, idxb0, idxb1, sem0, sem1):
        core = lax.axis_index("c")
        sid = lax.axis_index("s")
        rbase = sid * ROWS_PER_SUB

        for k in range(N_CHUNKS):
            @pl.when(core == (k % SC_CORES))
            def _(k=k):
                pltpu.sync_copy(z_hbm, acc.at[pl.ds(rbase, ROWS_PER_SUB)])
                plsc.subcore_barrier()

                def issue(g, buf, idxb, sem):
                    step = sid + g * SC_SUBCORES

                    @pl.when(step < SCAT_TOTAL_STEPS)
                    def _():
                        pltpu.async_copy(
                            eo_hbm.at[k, pl.ds(step * SCAT_E, SCAT_E)],
                            buf, sem)
                        pltpu.async_copy(ii_hbm.at[step], idxb, sem)

                def drain_add(g, buf, idxb, sem):
                    step = sid + g * SC_SUBCORES

                    @pl.when(step < SCAT_TOTAL_STEPS)
                    def _():
                        pltpu.make_async_copy(
                            eo_hbm.at[k, pl.ds(step * SCAT_E, SCAT_E)],
                            buf, sem).wait()
                        pltpu.make_async_copy(ii_hbm.at[step], idxb, sem).wait()
                        pltpu.sync_copy(buf, acc.at[idxb.at[0]], add=True)

                issue(0, buf0, idxb0, sem0)

                @pl.loop(0, SCAT_OUTER)
                def _(gi):
                    g0 = gi * 2
                    issue(g0 + 1, buf1, idxb1, sem1)
                    drain_add(g0, buf0, idxb0, sem0)
                    issue(g0 + 2, buf0, idxb0, sem0)
                    drain_add(g0 + 1, buf1, idxb1, sem1)

                plsc.subcore_barrier()
                pltpu.sync_copy(
                    acc.at[pl.ds(rbase, ROWS_PER_SUB)],
                    out_hbm.at[k, pl.ds(rbase, ROWS_PER_SUB)])

    return scatter_kernel(edge_out, idx_i3, zeros_rows)


def kernel(input_tensor_0, input_tensor_1, input_tensor_2, coordinate,
           W_r0, b_r0, W_r1, b_r1, W_r2, b_r2, idx_i, idx_j, atomic_number):
    n = N_ATOMS
    t1t = input_tensor_1.transpose(0, 2, 1).reshape(n, 3 * N_CHANNEL)
    t2t = input_tensor_2.transpose(0, 2, 3, 1).reshape(n, 9 * N_CHANNEL)
    coordp = jnp.pad(coordinate, ((0, 0), (0, D_COORD - 3)))
    table = jnp.concatenate([input_tensor_0, t1t, t2t, coordp], axis=1)

    gathered, ci = _gather_stage(table, coordp, idx_j, idx_i)
    edge_out = _tc_stage(gathered, ci, W_r0, b_r0, W_r1, b_r1, W_r2, b_r2)
    zeros_rows = jnp.zeros((ROWS_PER_SUB, N_CHANNEL), jnp.float32)
    out_flat = _scatter_stage(edge_out, idx_i, zeros_rows)

    out_flat = out_flat[:, :n]
    out0 = out_flat[0]
    out1 = out_flat[1:4].transpose(1, 2, 0)
    out2 = out_flat[4:13].reshape(3, 3, n, N_CHANNEL).transpose(2, 3, 0, 1)
    return (out0, out1, out2)


# speedup vs baseline: 63.4913x; 1.1175x over previous
import dataclasses
import functools

import jax
import jax.numpy as jnp
from jax import lax
from jax.experimental import pallas as pl
from jax.experimental.pallas import tpu as pltpu
from jax.experimental.pallas import tpu_sc as plsc

N_ATOMS = 10000
N_EDGES = 160000
N_CHANNEL = 128
N_RBF = 16
R_CUT = 12.0
NORM_FACTOR = 16.0

D_TABLE = 13 * N_CHANNEL
D_AUX = 8
D_OUT = 13 * N_CHANNEL

SC_CORES = 2
SC_SUBCORES = 16
SC_WORKERS = SC_CORES * SC_SUBCORES

GW = 24
E_PAD = 161280
N_GBLK = E_PAD // GW

N_CHUNKS = 13
EDGES_PER_SUB = N_EDGES // SC_SUBCORES
SCAT_E = 128
SCAT_TOTAL_STEPS = N_EDGES // SCAT_E
SCAT_OUTER = 40
N_PAD = 10240
ROWS_PER_SUB = N_PAD // SC_SUBCORES

TC_B = 640


def _sc_mesh():
    return plsc.VectorSubcoreMesh(core_axis_name="c", subcore_axis_name="s")


def _sc_no_layout_params():
    cp = pltpu.CompilerParams()
    if "needs_layout_passes" in pltpu.CompilerParams.__dataclass_fields__:
        cp = dataclasses.replace(cp, needs_layout_passes=False)
    return cp


def _gather_stage(table, cx, cy, cz, idx_j, idx_i):
    pad = E_PAD - N_EDGES
    idx_j3 = jnp.pad(idx_j, (0, pad)).reshape(N_GBLK, 1, GW)
    idx_i3 = jnp.pad(idx_i, (0, pad)).reshape(N_GBLK, 1, GW)

    @functools.partial(
        pl.kernel,
        out_type=[
            jax.ShapeDtypeStruct((E_PAD, D_TABLE), jnp.float32),
            jax.ShapeDtypeStruct((E_PAD, D_AUX), jnp.float32),
        ],
        mesh=_sc_mesh(),
        compiler_params=_sc_no_layout_params(),
        scratch_types=[
            pltpu.VMEM((N_ATOMS,), jnp.float32),
            pltpu.VMEM((N_ATOMS,), jnp.float32),
            pltpu.VMEM((N_ATOMS,), jnp.float32),
        ],
    )
    def gather_kernel(table_hbm, cx_hbm, cy_hbm, cz_hbm, ij_hbm, ii_hbm,
                      g_hbm, aux_hbm, cx_v, cy_v, cz_v):
        pltpu.sync_copy(cx_hbm, cx_v)
        pltpu.sync_copy(cy_hbm, cy_v)
        pltpu.sync_copy(cz_hbm, cz_v)

        def body(ij_vmem, ii_vmem, g_vmem, aux_vmem):
            pltpu.sync_copy(table_hbm.at[ij_vmem.at[0, 0]], g_vmem)
            for off in (0, GW - 16):
                ij = ij_vmem[0, 0, pl.ds(off, 16)]
                ii = ii_vmem[0, 0, pl.ds(off, 16)]
                rows = lax.iota(jnp.int32, 16) + off
                for c, cref in enumerate((cx_v, cy_v, cz_v)):
                    rc = (plsc.load_gather(cref, [ij])
                          - plsc.load_gather(cref, [ii]))
                    cols = jnp.full((16,), c, jnp.int32)
                    plsc.store_scatter(aux_vmem, [rows, cols], rc)

        pltpu.emit_pipeline(
            body,
            grid=(N_GBLK,),
            in_specs=[
                pl.BlockSpec((1, 1, GW), lambda i: (i, 0, 0)),
                pl.BlockSpec((1, 1, GW), lambda i: (i, 0, 0)),
            ],
            out_specs=[
                pl.BlockSpec((GW, D_TABLE), lambda i: (i, 0)),
                pl.BlockSpec((GW, D_AUX), lambda i: (i, 0)),
            ],
            core_axis_name=("c", "s"),
            dimension_semantics=(pltpu.PARALLEL,),
        )(ij_hbm, ii_hbm, g_hbm, aux_hbm)

    return gather_kernel(table, cx, cy, cz, idx_j3, idx_i3)


def _sin_turns(x):
    r = jnp.round(x)
    u = 2.0 * (x - r)
    w = u * u
    p = jnp.float32(-0.00614081537025124)
    p = p * w + jnp.float32(0.08086610273610417)
    p = p * w + jnp.float32(-0.5986449462113246)
    p = p * w + jnp.float32(2.5500285525212325)
    p = p * w + jnp.float32(-5.167702003410101)
    p = p * w + jnp.float32(3.1415925159778033)
    return p * u


def _tc_body(g_ref, aux_ref, w0_ref, b0_ref, w1_ref, b1_ref, w2_ref, b2_ref,
             out_ref):
    g = g_ref[...]
    r = aux_ref[:, 0:3]
    d2 = jnp.sum(r * r, axis=1, keepdims=True) + 1e-12
    d = jnp.sqrt(d2)
    u = r / d

    n = (lax.broadcasted_iota(jnp.int32, (1, N_RBF), 1) + 1).astype(jnp.float32)
    xs = d * (n * (0.5 / R_CUT))
    xc = d * (0.5 / R_CUT) + 0.25
    sn = _sin_turns(xs)
    c1 = _sin_turns(xc)
    fc = jnp.where(d < R_CUT, 0.5 * (c1 + 1.0), 0.0)
    rbf = sn * (fc / d)

    inv_norm = 1.0 / NORM_FACTOR
    f0 = (jnp.dot(rbf, w0_ref[...], preferred_element_type=jnp.float32)
          + b0_ref[...]) * inv_norm
    f1 = (jnp.dot(rbf, w1_ref[...], preferred_element_type=jnp.float32)
          + b1_ref[...]) * inv_norm
    f2 = (jnp.dot(rbf, w2_ref[...], preferred_element_type=jnp.float32)
          + b2_ref[...]) * inv_norm

    C = N_CHANNEL
    g0 = g[:, 0:C]
    g1 = [g[:, C + a * C: C + (a + 1) * C] for a in range(3)]
    g2 = [[g[:, 4 * C + (3 * a + b) * C: 4 * C + (3 * a + b + 1) * C]
           for b in range(3)] for a in range(3)]
    ub = [u[:, a:a + 1] for a in range(3)]

    d1 = g1[0] * ub[0] + g1[1] * ub[1] + g1[2] * ub[2]
    v2 = [g2[0][b] * ub[0] + g2[1][b] * ub[1] + g2[2][b] * ub[2]
          for b in range(3)]
    q2 = v2[0] * ub[0] + v2[1] * ub[1] + v2[2] * ub[2]

    out_ref[0, :, :] = f0 * g0 + f1 * d1 + f2 * q2
    f1g0 = f1 * g0
    f2d1 = f2 * d1
    for b in range(3):
        out_ref[1 + b, :, :] = (
            f0 * g1[b] + f1g0 * ub[b] + f1 * v2[b] + f2d1 * ub[b])
    f2g0 = f2 * g0
    for a in range(3):
        ta = f2g0 * ub[a] + f1 * g1[a] + f2 * v2[a]
        for b in range(3):
            out_ref[4 + 3 * a + b, :, :] = f0 * g2[a][b] + ta * ub[b]


def _tc_stage(gathered, aux, w0, b0, w1, b1, w2, b2):
    const_w = pl.BlockSpec((N_RBF, N_CHANNEL), lambda i: (0, 0))
    const_b = pl.BlockSpec((1, N_CHANNEL), lambda i: (0, 0))
    return pl.pallas_call(
        _tc_body,
        grid=(E_PAD // TC_B,),
        in_specs=[
            pl.BlockSpec((TC_B, D_TABLE), lambda i: (i, 0)),
            pl.BlockSpec((TC_B, D_AUX), lambda i: (i, 0)),
            const_w, const_b, const_w, const_b, const_w, const_b,
        ],
        out_specs=pl.BlockSpec((N_CHUNKS, TC_B, N_CHANNEL), lambda i: (0, i, 0)),
        out_shape=jax.ShapeDtypeStruct((N_CHUNKS, E_PAD, N_CHANNEL), jnp.float32),
    )(gathered, aux, w0, b0.reshape(1, N_CHANNEL), w1,
      b1.reshape(1, N_CHANNEL), w2, b2.reshape(1, N_CHANNEL))


def _scatter_stage(edge_out, idx_i, zeros_rows):
    idx_i3 = idx_i.reshape(N_EDGES // SCAT_E, 1, SCAT_E)

    @functools.partial(
        pl.kernel,
        out_type=jax.ShapeDtypeStruct((N_CHUNKS, N_PAD, N_CHANNEL), jnp.float32),
        mesh=_sc_mesh(),
        scratch_types=[
            pltpu.VMEM_SHARED((N_PAD, N_CHANNEL), jnp.float32),
            pltpu.VMEM((SCAT_E, N_CHANNEL), jnp.float32),
            pltpu.VMEM((SCAT_E, N_CHANNEL), jnp.float32),
            pltpu.VMEM((1, SCAT_E), jnp.int32),
            pltpu.VMEM((1, SCAT_E), jnp.int32),
            pltpu.SemaphoreType.DMA,
            pltpu.SemaphoreType.DMA,
        ],
    )
    def scatter_kernel(eo_hbm, ii_hbm, z_hbm, out_hbm, acc,
                       buf0, buf1, idxb0, idxb1, sem0, sem1):
        core = lax.axis_index("c")
        sid = lax.axis_index("s")
        rbase = sid * ROWS_PER_SUB

        for k in range(N_CHUNKS):
            @pl.when(core == (k % SC_CORES))
            def _(k=k):
                pltpu.sync_copy(z_hbm, acc.at[pl.ds(rbase, ROWS_PER_SUB)])
                plsc.subcore_barrier()

                def issue(g, buf, idxb, sem):
                    step = sid + g * SC_SUBCORES

                    @pl.when(step < SCAT_TOTAL_STEPS)
                    def _():
                        pltpu.async_copy(
                            eo_hbm.at[k, pl.ds(step * SCAT_E, SCAT_E)],
                            buf, sem)
                        pltpu.async_copy(ii_hbm.at[step], idxb, sem)

                def drain_add(g, buf, idxb, sem):
                    step = sid + g * SC_SUBCORES

                    @pl.when(step < SCAT_TOTAL_STEPS)
                    def _():
                        pltpu.make_async_copy(
                            eo_hbm.at[k, pl.ds(step * SCAT_E, SCAT_E)],
                            buf, sem).wait()
                        pltpu.make_async_copy(ii_hbm.at[step], idxb, sem).wait()
                        pltpu.sync_copy(buf, acc.at[idxb.at[0]], add=True)

                issue(0, buf0, idxb0, sem0)

                @pl.loop(0, SCAT_OUTER)
                def _(gi):
                    g0 = gi * 2
                    issue(g0 + 1, buf1, idxb1, sem1)
                    drain_add(g0, buf0, idxb0, sem0)
                    issue(g0 + 2, buf0, idxb0, sem0)
                    drain_add(g0 + 1, buf1, idxb1, sem1)

                plsc.subcore_barrier()
                pltpu.sync_copy(
                    acc.at[pl.ds(rbase, ROWS_PER_SUB)],
                    out_hbm.at[k, pl.ds(rbase, ROWS_PER_SUB)])

    return scatter_kernel(edge_out, idx_i3, zeros_rows)


def kernel(input_tensor_0, input_tensor_1, input_tensor_2, coordinate,
           W_r0, b_r0, W_r1, b_r1, W_r2, b_r2, idx_i, idx_j, atomic_number):
    n = N_ATOMS
    t1t = input_tensor_1.transpose(0, 2, 1).reshape(n, 3 * N_CHANNEL)
    t2t = input_tensor_2.transpose(0, 2, 3, 1).reshape(n, 9 * N_CHANNEL)
    table = jnp.concatenate([input_tensor_0, t1t, t2t], axis=1)

    gathered, aux = _gather_stage(table, coordinate[:, 0], coordinate[:, 1],
                                  coordinate[:, 2], idx_j, idx_i)
    edge_out = _tc_stage(gathered, aux, W_r0, b_r0, W_r1, b_r1, W_r2, b_r2)
    zeros_rows = jnp.zeros((ROWS_PER_SUB, N_CHANNEL), jnp.float32)
    out_flat = _scatter_stage(edge_out, idx_i, zeros_rows)

    out_flat = out_flat[:, :n]
    out0 = out_flat[0]
    out1 = out_flat[1:4].transpose(1, 2, 0)
    out2 = out_flat[4:13].reshape(3, 3, n, N_CHANNEL).transpose(2, 3, 0, 1)
    return (out0, out1, out2)


# speedup vs baseline: 64.4303x; 1.0148x over previous
import dataclasses
import functools

import jax
import jax.numpy as jnp
from jax import lax
from jax.experimental import pallas as pl
from jax.experimental.pallas import tpu as pltpu
from jax.experimental.pallas import tpu_sc as plsc

N_ATOMS = 10000
N_EDGES = 160000
N_CHANNEL = 128
N_RBF = 16
R_CUT = 12.0
NORM_FACTOR = 16.0

D_TABLE = 13 * N_CHANNEL
D_AUX = 8
D_OUT = 13 * N_CHANNEL

SC_CORES = 2
SC_SUBCORES = 16
SC_WORKERS = SC_CORES * SC_SUBCORES

GW = 24
E_PAD = 161280
N_GBLK = E_PAD // GW

N_CHUNKS = 13
EDGES_PER_SUB = N_EDGES // SC_SUBCORES
SCAT_E = 128
SCAT_TOTAL_STEPS = N_EDGES // SCAT_E
SCAT_OUTER = 40
N_PAD = 10240
ROWS_PER_SUB = N_PAD // SC_SUBCORES

TC_B = 640


def _sc_mesh():
    return plsc.VectorSubcoreMesh(core_axis_name="c", subcore_axis_name="s")


def _sc_no_layout_params():
    cp = pltpu.CompilerParams()
    if "needs_layout_passes" in pltpu.CompilerParams.__dataclass_fields__:
        cp = dataclasses.replace(cp, needs_layout_passes=False)
    return cp


def _gather_stage(table, cx, cy, cz, idx_j, idx_i):
    n_rows = idx_j.shape[0] * GW

    @functools.partial(
        pl.kernel,
        out_type=[
            jax.ShapeDtypeStruct((n_rows, D_TABLE), jnp.float32),
            jax.ShapeDtypeStruct((n_rows, D_AUX), jnp.float32),
        ],
        mesh=_sc_mesh(),
        compiler_params=_sc_no_layout_params(),
        scratch_types=[
            pltpu.VMEM((N_ATOMS,), jnp.float32),
            pltpu.VMEM((N_ATOMS,), jnp.float32),
            pltpu.VMEM((N_ATOMS,), jnp.float32),
        ],
    )
    def gather_kernel(table_hbm, cx_hbm, cy_hbm, cz_hbm, ij_hbm, ii_hbm,
                      g_hbm, aux_hbm, cx_v, cy_v, cz_v):
        pltpu.sync_copy(cx_hbm, cx_v)
        pltpu.sync_copy(cy_hbm, cy_v)
        pltpu.sync_copy(cz_hbm, cz_v)

        def body(ij_vmem, ii_vmem, g_vmem, aux_vmem):
            pltpu.sync_copy(table_hbm.at[ij_vmem.at[0, 0]], g_vmem)
            for off in (0, GW - 16):
                ij = ij_vmem[0, 0, pl.ds(off, 16)]
                ii = ii_vmem[0, 0, pl.ds(off, 16)]
                rows = lax.iota(jnp.int32, 16) + off
                for c, cref in enumerate((cx_v, cy_v, cz_v)):
                    rc = (plsc.load_gather(cref, [ij])
                          - plsc.load_gather(cref, [ii]))
                    cols = jnp.full((16,), c, jnp.int32)
                    plsc.store_scatter(aux_vmem, [rows, cols], rc)

        pltpu.emit_pipeline(
            body,
            grid=(n_rows // GW,),
            in_specs=[
                pl.BlockSpec((1, 1, GW), lambda i: (i, 0, 0)),
                pl.BlockSpec((1, 1, GW), lambda i: (i, 0, 0)),
            ],
            out_specs=[
                pl.BlockSpec((GW, D_TABLE), lambda i: (i, 0)),
                pl.BlockSpec((GW, D_AUX), lambda i: (i, 0)),
            ],
            core_axis_name=("c", "s"),
            dimension_semantics=(pltpu.PARALLEL,),
        )(ij_hbm, ii_hbm, g_hbm, aux_hbm)

    return gather_kernel(table, cx, cy, cz, idx_j, idx_i)


def _sin_turns(x):
    r = jnp.round(x)
    u = 2.0 * (x - r)
    w = u * u
    p = jnp.float32(-0.00614081537025124)
    p = p * w + jnp.float32(0.08086610273610417)
    p = p * w + jnp.float32(-0.5986449462113246)
    p = p * w + jnp.float32(2.5500285525212325)
    p = p * w + jnp.float32(-5.167702003410101)
    p = p * w + jnp.float32(3.1415925159778033)
    return p * u


def _tc_body(g_ref, aux_ref, w0_ref, b0_ref, w1_ref, b1_ref, w2_ref, b2_ref,
             out_ref):
    g = g_ref[...]
    r = aux_ref[:, 0:3]
    d2 = jnp.sum(r * r, axis=1, keepdims=True) + 1e-12
    d = jnp.sqrt(d2)
    u = r / d

    n = (lax.broadcasted_iota(jnp.int32, (1, N_RBF), 1) + 1).astype(jnp.float32)
    xs = d * (n * (0.5 / R_CUT))
    xc = d * (0.5 / R_CUT) + 0.25
    sn = _sin_turns(xs)
    c1 = _sin_turns(xc)
    fc = jnp.where(d < R_CUT, 0.5 * (c1 + 1.0), 0.0)
    rbf = sn * (fc / d)

    inv_norm = 1.0 / NORM_FACTOR
    f0 = (jnp.dot(rbf, w0_ref[...], preferred_element_type=jnp.float32)
          + b0_ref[...]) * inv_norm
    f1 = (jnp.dot(rbf, w1_ref[...], preferred_element_type=jnp.float32)
          + b1_ref[...]) * inv_norm
    f2 = (jnp.dot(rbf, w2_ref[...], preferred_element_type=jnp.float32)
          + b2_ref[...]) * inv_norm

    C = N_CHANNEL
    g0 = g[:, 0:C]
    g1 = [g[:, C + a * C: C + (a + 1) * C] for a in range(3)]
    g2 = [[g[:, 4 * C + (3 * a + b) * C: 4 * C + (3 * a + b + 1) * C]
           for b in range(3)] for a in range(3)]
    ub = [u[:, a:a + 1] for a in range(3)]

    d1 = g1[0] * ub[0] + g1[1] * ub[1] + g1[2] * ub[2]
    v2 = [g2[0][b] * ub[0] + g2[1][b] * ub[1] + g2[2][b] * ub[2]
          for b in range(3)]
    q2 = v2[0] * ub[0] + v2[1] * ub[1] + v2[2] * ub[2]

    out_ref[0, :, :] = f0 * g0 + f1 * d1 + f2 * q2
    f1g0 = f1 * g0
    f2d1 = f2 * d1
    for b in range(3):
        out_ref[1 + b, :, :] = (
            f0 * g1[b] + f1g0 * ub[b] + f1 * v2[b] + f2d1 * ub[b])
    f2g0 = f2 * g0
    for a in range(3):
        ta = f2g0 * ub[a] + f1 * g1[a] + f2 * v2[a]
        for b in range(3):
            out_ref[4 + 3 * a + b, :, :] = f0 * g2[a][b] + ta * ub[b]


def _tc_stage(gathered, aux, w0, b0, w1, b1, w2, b2):
    const_w = pl.BlockSpec((N_RBF, N_CHANNEL), lambda i: (0, 0))
    const_b = pl.BlockSpec((1, N_CHANNEL), lambda i: (0, 0))
    n_rows = gathered.shape[0]
    return pl.pallas_call(
        _tc_body,
        grid=(n_rows // TC_B,),
        in_specs=[
            pl.BlockSpec((TC_B, D_TABLE), lambda i: (i, 0)),
            pl.BlockSpec((TC_B, D_AUX), lambda i: (i, 0)),
            const_w, const_b, const_w, const_b, const_w, const_b,
        ],
        out_specs=pl.BlockSpec((N_CHUNKS, TC_B, N_CHANNEL), lambda i: (0, i, 0)),
        out_shape=jax.ShapeDtypeStruct((N_CHUNKS, n_rows, N_CHANNEL), jnp.float32),
    )(gathered, aux, w0, b0.reshape(1, N_CHANNEL), w1,
      b1.reshape(1, N_CHANNEL), w2, b2.reshape(1, N_CHANNEL))


E_HALF = E_PAD // 2
STEPS_A = E_HALF // SCAT_E
STEP_MID = SCAT_TOTAL_STEPS // 2


def _scatter_stage(eo_a, eo_b, idx_i, zeros_rows):
    idx_i3 = idx_i.reshape(N_EDGES // SCAT_E, 1, SCAT_E)

    @functools.partial(
        pl.kernel,
        out_type=jax.ShapeDtypeStruct((N_CHUNKS + 1, N_PAD, N_CHANNEL),
                                      jnp.float32),
        mesh=_sc_mesh(),
        scratch_types=[
            pltpu.VMEM_SHARED((N_PAD, N_CHANNEL), jnp.float32),
            pltpu.VMEM((SCAT_E, N_CHANNEL), jnp.float32),
            pltpu.VMEM((SCAT_E, N_CHANNEL), jnp.float32),
            pltpu.VMEM((1, SCAT_E), jnp.int32),
            pltpu.VMEM((1, SCAT_E), jnp.int32),
            pltpu.SemaphoreType.DMA,
            pltpu.SemaphoreType.DMA,
        ],
    )
    def scatter_kernel(eoa_hbm, eob_hbm, ii_hbm, z_hbm, out_hbm, acc,
                       buf0, buf1, idxb0, idxb1, sem0, sem1):
        core = lax.axis_index("c")
        sid = lax.axis_index("s")
        rbase = sid * ROWS_PER_SUB

        def do_chunk(k, out_slot, step_lo, step_hi):
            pltpu.sync_copy(z_hbm, acc.at[pl.ds(rbase, ROWS_PER_SUB)])
            plsc.subcore_barrier()

            def issue(g, buf, idxb, sem):
                step = step_lo + sid + g * SC_SUBCORES

                @pl.when(step < step_hi)
                def _():
                    @pl.when(step < STEPS_A)
                    def _():
                        pltpu.async_copy(
                            eoa_hbm.at[k, pl.ds(step * SCAT_E, SCAT_E)],
                            buf, sem)

                    @pl.when(step >= STEPS_A)
                    def _():
                        pltpu.async_copy(
                            eob_hbm.at[k, pl.ds((step - STEPS_A) * SCAT_E,
                                                SCAT_E)],
                            buf, sem)

                    pltpu.async_copy(ii_hbm.at[step], idxb, sem)

            def drain_add(g, buf, idxb, sem):
                step = step_lo + sid + g * SC_SUBCORES

                @pl.when(step < step_hi)
                def _():
                    pltpu.make_async_copy(
                        eoa_hbm.at[k, pl.ds(0, SCAT_E)], buf, sem).wait()
                    pltpu.make_async_copy(ii_hbm.at[step], idxb, sem).wait()
                    pltpu.sync_copy(buf, acc.at[idxb.at[0]], add=True)

            issue(0, buf0, idxb0, sem0)

            @pl.loop(0, SCAT_OUTER)
            def _(gi):
                g0 = gi * 2
                issue(g0 + 1, buf1, idxb1, sem1)
                drain_add(g0, buf0, idxb0, sem0)
                issue(g0 + 2, buf0, idxb0, sem0)
                drain_add(g0 + 1, buf1, idxb1, sem1)

            plsc.subcore_barrier()
            pltpu.sync_copy(
                acc.at[pl.ds(rbase, ROWS_PER_SUB)],
                out_hbm.at[out_slot, pl.ds(rbase, ROWS_PER_SUB)])

        for k in range(12):
            @pl.when(core == (k % SC_CORES))
            def _(k=k):
                do_chunk(k, k, 0, SCAT_TOTAL_STEPS)

        @pl.when(core == 0)
        def _():
            do_chunk(12, 12, 0, STEP_MID)

        @pl.when(core == 1)
        def _():
            do_chunk(12, 13, STEP_MID, SCAT_TOTAL_STEPS)

    return scatter_kernel(eo_a, eo_b, idx_i3, zeros_rows)


def kernel(input_tensor_0, input_tensor_1, input_tensor_2, coordinate,
           W_r0, b_r0, W_r1, b_r1, W_r2, b_r2, idx_i, idx_j, atomic_number):
    n = N_ATOMS
    t1t = input_tensor_1.transpose(0, 2, 1).reshape(n, 3 * N_CHANNEL)
    t2t = input_tensor_2.transpose(0, 2, 3, 1).reshape(n, 9 * N_CHANNEL)
    table = jnp.concatenate([input_tensor_0, t1t, t2t], axis=1)

    pad = E_PAD - N_EDGES
    idx_j3 = jnp.pad(idx_j, (0, pad)).reshape(N_GBLK, 1, GW)
    idx_i3 = jnp.pad(idx_i, (0, pad)).reshape(N_GBLK, 1, GW)
    cx, cy, cz = coordinate[:, 0], coordinate[:, 1], coordinate[:, 2]
    h = N_GBLK // 2

    g_a, aux_a = _gather_stage(table, cx, cy, cz, idx_j3[:h], idx_i3[:h])
    eo_a = _tc_stage(g_a, aux_a, W_r0, b_r0, W_r1, b_r1, W_r2, b_r2)
    g_b, aux_b = _gather_stage(table, cx, cy, cz, idx_j3[h:], idx_i3[h:])
    eo_b = _tc_stage(g_b, aux_b, W_r0, b_r0, W_r1, b_r1, W_r2, b_r2)

    zeros_rows = jnp.zeros((ROWS_PER_SUB, N_CHANNEL), jnp.float32)
    out_raw = _scatter_stage(eo_a, eo_b, idx_i, zeros_rows)

    out_flat = out_raw[:, :n]
    out0 = out_flat[0]
    out1 = out_flat[1:4].transpose(1, 2, 0)
    o2 = jnp.concatenate(
        [out_flat[4:12], (out_flat[12] + out_flat[13])[None]], axis=0)
    out2 = o2.reshape(3, 3, n, N_CHANNEL).transpose(2, 3, 0, 1)
    return (out0, out1, out2)


# speedup vs baseline: 71.2247x; 1.1055x over previous
import dataclasses
import functools

import jax
import jax.numpy as jnp
from jax import lax
from jax.experimental import pallas as pl
from jax.experimental.pallas import tpu as pltpu
from jax.experimental.pallas import tpu_sc as plsc

N_ATOMS = 10000
N_EDGES = 160000
N_CHANNEL = 128
N_RBF = 16
R_CUT = 12.0
NORM_FACTOR = 16.0

D_PACK = 7 * N_CHANNEL
D_AUX = 8
D_OUT = 13 * N_CHANNEL

SC_CORES = 2
SC_SUBCORES = 16
SC_WORKERS = SC_CORES * SC_SUBCORES

GW = 48
E_PAD = 162816
N_GBLK = E_PAD // GW

N_CHUNKS = 13
EDGES_PER_SUB = N_EDGES // SC_SUBCORES
SCAT_E = 128
SCAT_TOTAL_STEPS = N_EDGES // SCAT_E
SCAT_OUTER = 40
N_PAD = 10240
ROWS_PER_SUB = N_PAD // SC_SUBCORES

TC_B = 512


def _sc_mesh():
    return plsc.VectorSubcoreMesh(core_axis_name="c", subcore_axis_name="s")


def _sc_no_layout_params():
    cp = pltpu.CompilerParams()
    if "needs_layout_passes" in pltpu.CompilerParams.__dataclass_fields__:
        cp = dataclasses.replace(cp, needs_layout_passes=False)
    return cp


def _gather_stage(table, cx, cy, cz, idx_j, idx_i):
    n_rows = idx_j.shape[0] * GW

    @functools.partial(
        pl.kernel,
        out_type=[
            jax.ShapeDtypeStruct((n_rows, D_PACK), jnp.int32),
            jax.ShapeDtypeStruct((n_rows, D_AUX), jnp.float32),
        ],
        mesh=_sc_mesh(),
        compiler_params=_sc_no_layout_params(),
        scratch_types=[
            pltpu.VMEM((N_ATOMS,), jnp.float32),
            pltpu.VMEM((N_ATOMS,), jnp.float32),
            pltpu.VMEM((N_ATOMS,), jnp.float32),
        ],
    )
    def gather_kernel(table_hbm, cx_hbm, cy_hbm, cz_hbm, ij_hbm, ii_hbm,
                      g_hbm, aux_hbm, cx_v, cy_v, cz_v):
        pltpu.sync_copy(cx_hbm, cx_v)
        pltpu.sync_copy(cy_hbm, cy_v)
        pltpu.sync_copy(cz_hbm, cz_v)

        def body(ij_vmem, ii_vmem, g_vmem, aux_vmem):
            pltpu.sync_copy(table_hbm.at[ij_vmem.at[0, 0]], g_vmem)
            for off in range(0, GW, 16):
                ij = ij_vmem[0, 0, pl.ds(off, 16)]
                ii = ii_vmem[0, 0, pl.ds(off, 16)]
                rows = lax.iota(jnp.int32, 16) + off
                for c, cref in enumerate((cx_v, cy_v, cz_v)):
                    rc = (plsc.load_gather(cref, [ij])
                          - plsc.load_gather(cref, [ii]))
                    cols = jnp.full((16,), c, jnp.int32)
                    plsc.store_scatter(aux_vmem, [rows, cols], rc)

        pltpu.emit_pipeline(
            body,
            grid=(n_rows // GW,),
            in_specs=[
                pl.BlockSpec((1, 1, GW), lambda i: (i, 0, 0)),
                pl.BlockSpec((1, 1, GW), lambda i: (i, 0, 0)),
            ],
            out_specs=[
                pl.BlockSpec((GW, D_PACK), lambda i: (i, 0)),
                pl.BlockSpec((GW, D_AUX), lambda i: (i, 0)),
            ],
            core_axis_name=("c", "s"),
            dimension_semantics=(pltpu.PARALLEL,),
        )(ij_hbm, ii_hbm, g_hbm, aux_hbm)

    return gather_kernel(table, cx, cy, cz, idx_j, idx_i)


def _sin_turns(x):
    r = jnp.round(x)
    u = 2.0 * (x - r)
    w = u * u
    p = jnp.float32(-0.00614081537025124)
    p = p * w + jnp.float32(0.08086610273610417)
    p = p * w + jnp.float32(-0.5986449462113246)
    p = p * w + jnp.float32(2.5500285525212325)
    p = p * w + jnp.float32(-5.167702003410101)
    p = p * w + jnp.float32(3.1415925159778033)
    return p * u


def _tc_body(g_ref, aux_ref, w0_ref, b0_ref, w1_ref, b1_ref, w2_ref, b2_ref,
             out_ref):
    w = g_ref[...]
    gw = []
    for k in range(7):
        wk = w[:, k * N_CHANNEL:(k + 1) * N_CHANNEL]
        gw.append(lax.bitcast_convert_type(wk << 16, jnp.float32))
        gw.append(lax.bitcast_convert_type(wk & jnp.int32(-65536),
                                           jnp.float32))
    r = aux_ref[:, 0:3]
    d2 = jnp.sum(r * r, axis=1, keepdims=True) + 1e-12
    d = jnp.sqrt(d2)
    u = r / d

    n = (lax.broadcasted_iota(jnp.int32, (1, N_RBF), 1) + 1).astype(jnp.float32)
    xs = d * (n * (0.5 / R_CUT))
    xc = d * (0.5 / R_CUT) + 0.25
    sn = _sin_turns(xs)
    c1 = _sin_turns(xc)
    fc = jnp.where(d < R_CUT, 0.5 * (c1 + 1.0), 0.0)
    rbf = sn * (fc / d)

    inv_norm = 1.0 / NORM_FACTOR
    f0 = (jnp.dot(rbf, w0_ref[...], preferred_element_type=jnp.float32)
          + b0_ref[...]) * inv_norm
    f1 = (jnp.dot(rbf, w1_ref[...], preferred_element_type=jnp.float32)
          + b1_ref[...]) * inv_norm
    f2 = (jnp.dot(rbf, w2_ref[...], preferred_element_type=jnp.float32)
          + b2_ref[...]) * inv_norm

    g0 = gw[0]
    g1 = gw[1:4]
    g2 = [[gw[4 + 3 * a + b] for b in range(3)] for a in range(3)]
    ub = [u[:, a:a + 1] for a in range(3)]

    d1 = g1[0] * ub[0] + g1[1] * ub[1] + g1[2] * ub[2]
    v2 = [g2[0][b] * ub[0] + g2[1][b] * ub[1] + g2[2][b] * ub[2]
          for b in range(3)]
    q2 = v2[0] * ub[0] + v2[1] * ub[1] + v2[2] * ub[2]

    out_ref[0, :, :] = f0 * g0 + f1 * d1 + f2 * q2
    f1g0 = f1 * g0
    f2d1 = f2 * d1
    for b in range(3):
        out_ref[1 + b, :, :] = (
            f0 * g1[b] + f1g0 * ub[b] + f1 * v2[b] + f2d1 * ub[b])
    f2g0 = f2 * g0
    for a in range(3):
        ta = f2g0 * ub[a] + f1 * g1[a] + f2 * v2[a]
        for b in range(3):
            out_ref[4 + 3 * a + b, :, :] = f0 * g2[a][b] + ta * ub[b]


def _tc_stage(gathered, aux, w0, b0, w1, b1, w2, b2):
    const_w = pl.BlockSpec((N_RBF, N_CHANNEL), lambda i: (0, 0))
    const_b = pl.BlockSpec((1, N_CHANNEL), lambda i: (0, 0))
    n_rows = gathered.shape[0]
    return pl.pallas_call(
        _tc_body,
        grid=(n_rows // TC_B,),
        in_specs=[
            pl.BlockSpec((TC_B, D_PACK), lambda i: (i, 0)),
            pl.BlockSpec((TC_B, D_AUX), lambda i: (i, 0)),
            const_w, const_b, const_w, const_b, const_w, const_b,
        ],
        out_specs=pl.BlockSpec((N_CHUNKS, TC_B, N_CHANNEL), lambda i: (0, i, 0)),
        out_shape=jax.ShapeDtypeStruct((N_CHUNKS, n_rows, N_CHANNEL), jnp.float32),
    )(gathered, aux, w0, b0.reshape(1, N_CHANNEL), w1,
      b1.reshape(1, N_CHANNEL), w2, b2.reshape(1, N_CHANNEL))


E_HALF = E_PAD // 2
STEPS_A = E_HALF // SCAT_E
STEP_MID = SCAT_TOTAL_STEPS // 2


def _scatter_stage(eo_a, eo_b, idx_i, zeros_rows):
    idx_i3 = idx_i.reshape(N_EDGES // SCAT_E, 1, SCAT_E)

    @functools.partial(
        pl.kernel,
        out_type=jax.ShapeDtypeStruct((N_CHUNKS + 1, N_PAD, N_CHANNEL),
                                      jnp.float32),
        mesh=_sc_mesh(),
        scratch_types=[
            pltpu.VMEM_SHARED((N_PAD, N_CHANNEL), jnp.float32),
            pltpu.VMEM((SCAT_E, N_CHANNEL), jnp.float32),
            pltpu.VMEM((SCAT_E, N_CHANNEL), jnp.float32),
            pltpu.VMEM((1, SCAT_E), jnp.int32),
            pltpu.VMEM((1, SCAT_E), jnp.int32),
            pltpu.SemaphoreType.DMA,
            pltpu.SemaphoreType.DMA,
        ],
    )
    def scatter_kernel(eoa_hbm, eob_hbm, ii_hbm, z_hbm, out_hbm, acc,
                       buf0, buf1, idxb0, idxb1, sem0, sem1):
        core = lax.axis_index("c")
        sid = lax.axis_index("s")
        rbase = sid * ROWS_PER_SUB

        def do_chunk(k, out_slot, step_lo, step_hi):
            pltpu.sync_copy(z_hbm, acc.at[pl.ds(rbase, ROWS_PER_SUB)])
            plsc.subcore_barrier()

            def issue(g, buf, idxb, sem):
                step = step_lo + sid + g * SC_SUBCORES

                @pl.when(step < step_hi)
                def _():
                    @pl.when(step < STEPS_A)
                    def _():
                        pltpu.async_copy(
                            eoa_hbm.at[k, pl.ds(step * SCAT_E, SCAT_E)],
                            buf, sem)

                    @pl.when(step >= STEPS_A)
                    def _():
                        pltpu.async_copy(
                            eob_hbm.at[k, pl.ds((step - STEPS_A) * SCAT_E,
                                                SCAT_E)],
                            buf, sem)

                    pltpu.async_copy(ii_hbm.at[step], idxb, sem)

            def drain_add(g, buf, idxb, sem):
                step = step_lo + sid + g * SC_SUBCORES

                @pl.when(step < step_hi)
                def _():
                    pltpu.make_async_copy(
                        eoa_hbm.at[k, pl.ds(0, SCAT_E)], buf, sem).wait()
                    pltpu.make_async_copy(ii_hbm.at[step], idxb, sem).wait()
                    pltpu.sync_copy(buf, acc.at[idxb.at[0]], add=True)

            issue(0, buf0, idxb0, sem0)

            @pl.loop(0, SCAT_OUTER)
            def _(gi):
                g0 = gi * 2
                issue(g0 + 1, buf1, idxb1, sem1)
                drain_add(g0, buf0, idxb0, sem0)
                issue(g0 + 2, buf0, idxb0, sem0)
                drain_add(g0 + 1, buf1, idxb1, sem1)

            plsc.subcore_barrier()
            pltpu.sync_copy(
                acc.at[pl.ds(rbase, ROWS_PER_SUB)],
                out_hbm.at[out_slot, pl.ds(rbase, ROWS_PER_SUB)])

        for k in range(12):
            @pl.when(core == (k % SC_CORES))
            def _(k=k):
                do_chunk(k, k, 0, SCAT_TOTAL_STEPS)

        @pl.when(core == 0)
        def _():
            do_chunk(12, 12, 0, STEP_MID)

        @pl.when(core == 1)
        def _():
            do_chunk(12, 13, STEP_MID, SCAT_TOTAL_STEPS)

    return scatter_kernel(eo_a, eo_b, idx_i3, zeros_rows)


def kernel(input_tensor_0, input_tensor_1, input_tensor_2, coordinate,
           W_r0, b_r0, W_r1, b_r1, W_r2, b_r2, idx_i, idx_j, atomic_number):
    n = N_ATOMS
    ways = jnp.concatenate(
        [input_tensor_0[:, None, :],
         input_tensor_1.transpose(0, 2, 1),
         input_tensor_2.transpose(0, 2, 3, 1).reshape(n, 9, N_CHANNEL),
         jnp.zeros((n, 1, N_CHANNEL), jnp.float32)],
        axis=1).astype(jnp.bfloat16)
    pairs = jnp.stack([ways[:, 0::2], ways[:, 1::2]], axis=-1)
    table = lax.bitcast_convert_type(pairs, jnp.int32).reshape(n, D_PACK)

    pad = E_PAD - N_EDGES
    idx_j3 = jnp.pad(idx_j, (0, pad)).reshape(N_GBLK, 1, GW)
    idx_i3 = jnp.pad(idx_i, (0, pad)).reshape(N_GBLK, 1, GW)
    cx, cy, cz = coordinate[:, 0], coordinate[:, 1], coordinate[:, 2]
    h = N_GBLK // 2

    g_a, aux_a = _gather_stage(table, cx, cy, cz, idx_j3[:h], idx_i3[:h])
    eo_a = _tc_stage(g_a, aux_a, W_r0, b_r0, W_r1, b_r1, W_r2, b_r2)
    g_b, aux_b = _gather_stage(table, cx, cy, cz, idx_j3[h:], idx_i3[h:])
    eo_b = _tc_stage(g_b, aux_b, W_r0, b_r0, W_r1, b_r1, W_r2, b_r2)

    zeros_rows = jnp.zeros((ROWS_PER_SUB, N_CHANNEL), jnp.float32)
    out_raw = _scatter_stage(eo_a, eo_b, idx_i, zeros_rows)

    out_flat = out_raw[:, :n]
    out0 = out_flat[0]
    out1 = out_flat[1:4].transpose(1, 2, 0)
    o2 = jnp.concatenate(
        [out_flat[4:12], (out_flat[12] + out_flat[13])[None]], axis=0)
    out2 = o2.reshape(3, 3, n, N_CHANNEL).transpose(2, 3, 0, 1)
    return (out0, out1, out2)
